# Initial kernel scaffold; baseline (speedup 1.0000x reference)
import jax
import jax.numpy as jnp
from jax.experimental import pallas as pl


def kernel(x, edge_index, W1, b1, W_mu, b_mu, W_logstd, b_logstd):
    raise NotImplementedError("write your pallas kernel here")


# speedup vs baseline: 12.3637x; 12.3637x over previous
import functools

import jax
import jax.numpy as jnp
from jax import lax
from jax.experimental import pallas as pl
from jax.experimental.pallas import tpu as pltpu
from jax.experimental.pallas import tpu_sc as plsc

N_NODES = 10000
D = 128
D_OUT = 64
NPAD = 10240
E_EDGES = 320000
CH = 128
EPAD = 323584
EPT = EPAD // 32
NCH = EPT // CH
RPT = NPAD // 16

_mesh = plsc.VectorSubcoreMesh(core_axis_name="c", subcore_axis_name="s")


@functools.partial(
    pl.kernel,
    out_type=jax.ShapeDtypeStruct((2, NPAD, D), jnp.float32),
    mesh=_mesh,
    scratch_types=[
        pltpu.VMEM((CH,), jnp.int32),
        pltpu.VMEM((CH,), jnp.int32),
        pltpu.VMEM((CH, D), jnp.float32),
        pltpu.VMEM_SHARED((NPAD, D), jnp.float32),
        pltpu.SemaphoreType.DMA,
    ],
)
def _prop(table, srcp, dstp, out, sidx, didx, rows, acc, sem):
    cid = lax.axis_index("c")
    sid = lax.axis_index("s")

    def _zero_rows(i, carry):
        for j in range(D // 16):
            rows[i, pl.ds(j * 16, 16)] = jnp.zeros((16,), jnp.float32)
        return carry

    lax.fori_loop(0, CH, _zero_rows, 0)
    for k in range(RPT // CH):
        pltpu.sync_copy(rows, acc.at[pl.ds((sid * (RPT // CH) + k) * CH, CH)])
    plsc.subcore_barrier()

    ebase = (cid * 16 + sid) * EPT

    def _step(i, carry):
        b = ebase + i * CH
        pltpu.sync_copy(srcp.at[pl.ds(b, CH)], sidx)
        pltpu.sync_copy(dstp.at[pl.ds(b, CH)], didx)
        pltpu.async_copy(table.at[sidx], rows, sem).wait()
        pltpu.sync_copy(rows, acc.at[didx], add=True)
        return carry

    lax.fori_loop(0, NCH, _step, 0)
    plsc.subcore_barrier()

    r0 = sid * RPT
    pltpu.sync_copy(acc.at[pl.ds(r0, RPT)], out.at[cid, pl.ds(r0, RPT)])


@functools.partial(
    pl.kernel,
    out_type=jax.ShapeDtypeStruct((2, NPAD, D), jnp.float32),
    mesh=_mesh,
    scratch_types=[
        pltpu.VMEM((CH,), jnp.int32),
        pltpu.VMEM((CH, D), jnp.float32),
        pltpu.VMEM_SHARED((NPAD, D), jnp.float32),
        pltpu.SemaphoreType.DMA,
    ],
)
def _degcount(dstp, out, didx, ones_b, dacc, sem):
    cid = lax.axis_index("c")
    sid = lax.axis_index("s")

    def _fill0(i, carry):
        for j in range(D // 16):
            ones_b[i, pl.ds(j * 16, 16)] = jnp.zeros((16,), jnp.float32)
        return carry

    lax.fori_loop(0, CH, _fill0, 0)
    for k in range(RPT // CH):
        pltpu.sync_copy(ones_b, dacc.at[pl.ds((sid * (RPT // CH) + k) * CH, CH)])

    def _fill1(i, carry):
        for j in range(D // 16):
            ones_b[i, pl.ds(j * 16, 16)] = jnp.ones((16,), jnp.float32)
        return carry

    lax.fori_loop(0, CH, _fill1, 0)
    plsc.subcore_barrier()

    ebase = (cid * 16 + sid) * EPT

    def _step(i, carry):
        b = ebase + i * CH
        pltpu.sync_copy(dstp.at[pl.ds(b, CH)], didx)
        pltpu.sync_copy(ones_b, dacc.at[didx], add=True)
        return carry

    lax.fori_loop(0, NCH, _step, 0)
    plsc.subcore_barrier()

    r0 = sid * RPT
    pltpu.sync_copy(dacc.at[pl.ds(r0, RPT)], out.at[cid, pl.ds(r0, RPT)])


BM = 512
GRID = NPAD // BM


def _k1_body(deg_ref, x_ref, xs_ref):
    deg = deg_ref[0] + deg_ref[1] + 1.0
    dinv = lax.rsqrt(deg)
    xs_ref[...] = x_ref[...] * dinv[:, None]


_k1 = pl.pallas_call(
    _k1_body,
    grid=(GRID,),
    in_specs=[
        pl.BlockSpec((2, BM), lambda i: (0, i)),
        pl.BlockSpec((BM, D), lambda i: (i, 0)),
    ],
    out_specs=pl.BlockSpec((BM, D), lambda i: (i, 0)),
    out_shape=jax.ShapeDtypeStruct((NPAD, D), jnp.float32),
)


def _k2_body(deg_ref, raw_ref, x_ref, w_ref, b_ref, h_ref, hs_ref):
    deg = deg_ref[0] + deg_ref[1] + 1.0
    dinv = lax.rsqrt(deg)
    agg = (raw_ref[0] + raw_ref[1]) * dinv[:, None] + x_ref[...] * (dinv * dinv)[:, None]
    h = jnp.maximum(
        jnp.dot(agg, w_ref[...], preferred_element_type=jnp.float32) + b_ref[...], 0.0
    )
    h_ref[...] = h
    hs_ref[...] = h * dinv[:, None]


_k2 = pl.pallas_call(
    _k2_body,
    grid=(GRID,),
    in_specs=[
        pl.BlockSpec((2, BM), lambda i: (0, i)),
        pl.BlockSpec((2, BM, D), lambda i: (0, i, 0)),
        pl.BlockSpec((BM, D), lambda i: (i, 0)),
        pl.BlockSpec((D, D), lambda i: (0, 0)),
        pl.BlockSpec((1, D), lambda i: (0, 0)),
    ],
    out_specs=[
        pl.BlockSpec((BM, D), lambda i: (i, 0)),
        pl.BlockSpec((BM, D), lambda i: (i, 0)),
    ],
    out_shape=[
        jax.ShapeDtypeStruct((NPAD, D), jnp.float32),
        jax.ShapeDtypeStruct((NPAD, D), jnp.float32),
    ],
)


def _k3_body(deg_ref, raw_ref, h_ref, w_ref, b_ref, o_ref):
    deg = deg_ref[0] + deg_ref[1] + 1.0
    dinv = lax.rsqrt(deg)
    agg = (raw_ref[0] + raw_ref[1]) * dinv[:, None] + h_ref[...] * (dinv * dinv)[:, None]
    o_ref[...] = jnp.dot(agg, w_ref[...], preferred_element_type=jnp.float32) + b_ref[...]


_k3 = pl.pallas_call(
    _k3_body,
    grid=(GRID,),
    in_specs=[
        pl.BlockSpec((2, BM), lambda i: (0, i)),
        pl.BlockSpec((2, BM, D), lambda i: (0, i, 0)),
        pl.BlockSpec((BM, D), lambda i: (i, 0)),
        pl.BlockSpec((D, D), lambda i: (0, 0)),
        pl.BlockSpec((1, D), lambda i: (0, 0)),
    ],
    out_specs=pl.BlockSpec((BM, D), lambda i: (i, 0)),
    out_shape=jax.ShapeDtypeStruct((NPAD, D), jnp.float32),
)


def kernel(x, edge_index, W1, b1, W_mu, b_mu, W_logstd, b_logstd):
    src = edge_index[0].astype(jnp.int32)
    dst = edge_index[1].astype(jnp.int32)
    pad = EPAD - E_EDGES
    srcp = jnp.concatenate([src, jnp.zeros((pad,), jnp.int32)])
    dstp = jnp.concatenate([dst, jnp.full((pad,), N_NODES, jnp.int32)])
    xpad = jnp.pad(x, ((0, NPAD - N_NODES), (0, 0)))

    degp = _degcount(dstp)
    deg2 = degp[:, :, 0]

    xs = _k1(deg2, xpad)
    raw1 = _prop(xs, srcp, dstp)
    h, hs = _k2(deg2, raw1, xpad, W1, b1.reshape(1, D))
    raw2 = _prop(hs, srcp, dstp)
    wcat = jnp.concatenate([W_mu, W_logstd], axis=1)
    bcat = jnp.concatenate([b_mu, b_logstd]).reshape(1, D)
    outc = _k3(deg2, raw2, h, wcat, bcat)
    return outc[:N_NODES, :D_OUT], outc[:N_NODES, D_OUT:]
